# Initial kernel scaffold; baseline (speedup 1.0000x reference)
import jax
import jax.numpy as jnp
from jax.experimental import pallas as pl


def kernel(x, z, x_grid, z_grid, latent, Wq, Wk, Wv, Wo):
    raise NotImplementedError("write your pallas kernel here")


# speedup vs baseline: 6.2322x; 6.2322x over previous
import functools
import math

import jax
import jax.numpy as jnp
from jax import lax
from jax.experimental import pallas as pl
from jax.experimental.pallas import tpu as pltpu
from jax.experimental.pallas import tpu_sc as plsc

EMBED = 256
HEADS = 8
HDIM = 32
MB = 4
NTOK = 4096
G1 = 64
G2 = 64
NGP = G1 * G2
DX = 2
BINBLK = 256
NLANE = 16
NCG = EMBED // NLANE
NSUB = 16
NWORK = 32
NITEM = MB * NCG
SCCHUNK = 512

_PREC = lax.Precision.HIGHEST
_f32 = jnp.float32
_i32 = jnp.int32

_NN = (((1,), (0,)), ((), ()))
_NT = (((1,), (1,)), ((), ()))
_TN = (((0,), (0,)), ((), ()))


def _head_select_T():
    dd = lax.broadcasted_iota(_i32, (EMBED, HEADS), 0)
    hh = lax.broadcasted_iota(_i32, (EMBED, HEADS), 1)
    return (dd // HDIM == hh).astype(_f32)


def _prea_body(xT_ref, z_ref, xgT_ref, zg_ref, lat_ref, wq_ref, wk_ref,
               idx_ref, den_ref, et_ref, eg_ref):
    xT = xT_ref[0]
    xgT = xgT_ref[0]
    z = z_ref[0]
    zg = zg_ref[0]
    lat = lat_ref[...]
    wq = wq_ref[...]
    wk = wk_ref[...]

    x0 = xT[0:1, :]
    x1 = xT[1:2, :]
    g0 = xgT[0:1, :]
    g1 = xgT[1:2, :]
    mn0 = jnp.min(g0, axis=1, keepdims=True)
    mx0 = jnp.max(g0, axis=1, keepdims=True)
    mn1 = jnp.min(g1, axis=1, keepdims=True)
    mx1 = jnp.max(g1, axis=1, keepdims=True)
    sp0 = (mx0 - mn0) / (G1 - 1.0)
    sp1 = (mx1 - mn1) / (G2 - 1.0)
    m0 = jnp.clip(jnp.floor((x0 - mn0 + 0.5 * sp0) / sp0), 0.0, G1 - 1.0)
    m1 = jnp.clip(jnp.floor((x1 - mn1 + 0.5 * sp1) / sp1), 0.0, G2 - 1.0)
    idx = (m0 * float(G2) + m1).astype(_i32)
    idx_ref[0] = idx

    q = jnp.dot(lat, wq, precision=_PREC)
    a_mat = jnp.dot(wk * q, _head_select_T(),
                    precision=_PREC) * (1.0 / math.sqrt(HDIM))

    _TT = (((0,), (1,)), ((), ()))
    lt = lax.dot_general(a_mat, z, _TT, precision=_PREC)
    lg = lax.dot_general(a_mat, zg, _TT, precision=_PREC)
    c = jnp.maximum(jnp.max(lt, axis=1, keepdims=True),
                    jnp.max(lg, axis=1, keepdims=True))
    et = jnp.exp(lt - c)
    eg = jnp.exp(lg - c)
    et_ref[0] = et
    eg_ref[0] = eg

    for bb in range(NGP // BINBLK):
        bins = lax.broadcasted_iota(_i32, (BINBLK, NTOK), 0) + bb * BINBLK
        maskf = (idx == bins).astype(_f32)
        den_blk = lax.dot_general(et, maskf, _NT, precision=_PREC)
        den_ref[0, :, pl.ds(bb * BINBLK, BINBLK)] = (
            den_blk + eg[:, bb * BINBLK:(bb + 1) * BINBLK])


def _full(shape):
    return pl.BlockSpec(shape, lambda m, _n=len(shape): (0,) * _n)


def _batch(shape):
    return pl.BlockSpec((1,) + shape, lambda m, _n=len(shape): (m,) + (0,) * _n)


def _prea_call(xT, z, xgT, zg, lat2, Wq, Wk):
    return pl.pallas_call(
        _prea_body,
        grid=(MB,),
        in_specs=[
            _batch((DX, NTOK)),
            _batch((NTOK, EMBED)),
            _batch((DX, NGP)),
            _batch((NGP, EMBED)),
            _full((1, EMBED)),
            _full((EMBED, EMBED)),
            _full((EMBED, EMBED)),
        ],
        out_specs=[
            _batch((1, NTOK)),
            _batch((HEADS, NGP)),
            _batch((HEADS, NTOK)),
            _batch((HEADS, NGP)),
        ],
        out_shape=[
            jax.ShapeDtypeStruct((MB, 1, NTOK), _i32),
            jax.ShapeDtypeStruct((MB, HEADS, NGP), _f32),
            jax.ShapeDtypeStruct((MB, HEADS, NTOK), _f32),
            jax.ShapeDtypeStruct((MB, HEADS, NGP), _f32),
        ],
    )(xT, z, xgT, zg, lat2, Wq, Wk)


def _preb_body(z_ref, e_ref, wv_ref, payload_ref):
    z = z_ref[0]
    e = e_ref[0]
    wv = wv_ref[...]
    vt = lax.dot_general(wv, z, (((0,), (1,)), ((), ())), precision=_PREC)
    ew = lax.dot_general(_head_select_T(), e, _NN, precision=_PREC)
    payload_ref[0] = (vt * ew).reshape(NCG, NLANE, NTOK)


def _preb_call(z, e, Wv):
    return pl.pallas_call(
        _preb_body,
        grid=(MB,),
        in_specs=[
            _batch((NTOK, EMBED)),
            _batch((HEADS, NTOK)),
            _full((EMBED, EMBED)),
        ],
        out_specs=_batch((NCG, NLANE, NTOK)),
        out_shape=jax.ShapeDtypeStruct((MB, NCG, NLANE, NTOK), _f32),
    )(z, e, Wv)


def _sc_body(tokp_hbm, gridp_hbm, idx_hbm, out_hbm, acc_v, stage_v, irow_v):
    c = lax.axis_index("c")
    s = lax.axis_index("s")
    wid = c * NSUB + s
    for it in range(NITEM // NWORK):
        item = wid + it * NWORK
        b = item // NCG
        cg = item % NCG
        pltpu.sync_copy(gridp_hbm.at[b, cg], acc_v)
        for t in range(NTOK // SCCHUNK):
            pltpu.sync_copy(tokp_hbm.at[b, cg, :, pl.ds(t * SCCHUNK, SCCHUNK)],
                            stage_v)
            pltpu.sync_copy(idx_hbm.at[b, :, pl.ds(t * SCCHUNK, SCCHUNK)],
                            irow_v)

            def step(i, carry):
                lane = lax.broadcasted_iota(_i32, (NLANE,), 0)
                fi = jnp.full((NLANE,), i, _i32)
                row = plsc.load_gather(irow_v, [lane, fi])
                val = plsc.load_gather(stage_v, [lane, fi])
                plsc.addupdate_scatter(acc_v, [lane, row], val)
                return carry

            lax.fori_loop(0, SCCHUNK, step, 0, unroll=8)
        pltpu.sync_copy(acc_v, out_hbm.at[b, cg])


@functools.cache
def _sc_scatter():
    return pl.kernel(
        _sc_body,
        out_type=jax.ShapeDtypeStruct((MB, NCG, NLANE, NGP), _f32),
        compiler_params=pltpu.CompilerParams(needs_layout_passes=False),
        mesh=plsc.VectorSubcoreMesh(core_axis_name="c", subcore_axis_name="s"),
        scratch_types=[
            pltpu.VMEM((NLANE, NGP), _f32),
            pltpu.VMEM((NLANE, SCCHUNK), _f32),
            pltpu.VMEM((NLANE, SCCHUNK), _i32),
        ],
    )


def _post_body(acc_ref, den_ref, wo_ref, out_ref):
    acc = acc_ref[0].reshape(EMBED, NGP)
    den = den_ref[0]
    den_wide = lax.dot_general(_head_select_T(), den, _NN,
                               precision=_PREC)
    y = acc / den_wide
    out_ref[0] = lax.dot_general(y, wo_ref[...], _TN,
                                 precision=_PREC)


def _post_call(acc, den, Wo):
    return pl.pallas_call(
        _post_body,
        grid=(MB,),
        in_specs=[
            pl.BlockSpec((1, NCG, NLANE, NGP), lambda m: (m, 0, 0, 0)),
            pl.BlockSpec((1, HEADS, NGP), lambda m: (m, 0, 0)),
            pl.BlockSpec((EMBED, EMBED), lambda m: (0, 0)),
        ],
        out_specs=pl.BlockSpec((1, NGP, EMBED), lambda m: (m, 0, 0)),
        out_shape=jax.ShapeDtypeStruct((MB, NGP, EMBED), _f32),
    )(acc, den, Wo)


def kernel(x, z, x_grid, z_grid, latent, Wq, Wk, Wv, Wo):
    xT = x.transpose(0, 2, 1)
    xgT = x_grid.reshape(MB, NGP, DX).transpose(0, 2, 1)
    zg_flat = z_grid.reshape(MB, NGP, EMBED)
    lat2 = latent.reshape(1, EMBED)

    idx3, den, et, eg = _prea_call(xT, z, xgT, zg_flat, lat2, Wq, Wk)
    tokp = _preb_call(z, et, Wv)
    gridp = _preb_call(zg_flat, eg, Wv)
    idx_rep = jnp.broadcast_to(idx3, (MB, NLANE, NTOK))

    acc = _sc_scatter()(tokp, gridp, idx_rep)

    zout = _post_call(acc, den, Wo)
    return (x_grid, zout.reshape(MB, G1, G2, EMBED))


# speedup vs baseline: 6.2328x; 1.0001x over previous
import functools
import math

import jax
import jax.numpy as jnp
from jax import lax
from jax.experimental import pallas as pl
from jax.experimental.pallas import tpu as pltpu
from jax.experimental.pallas import tpu_sc as plsc

EMBED = 256
HEADS = 8
HDIM = 32
MB = 4
NTOK = 4096
G1 = 64
G2 = 64
NGP = G1 * G2
DX = 2
BINBLK = 256
NLANE = 16
NCG = EMBED // NLANE
NSUB = 16
NWORK = 32
NITEM = MB * NCG
SCCHUNK = 512

_PREC = lax.Precision.HIGHEST
_f32 = jnp.float32
_i32 = jnp.int32

_NN = (((1,), (0,)), ((), ()))
_NT = (((1,), (1,)), ((), ()))
_TN = (((0,), (0,)), ((), ()))


def _head_select_T():
    dd = lax.broadcasted_iota(_i32, (EMBED, HEADS), 0)
    hh = lax.broadcasted_iota(_i32, (EMBED, HEADS), 1)
    return (dd // HDIM == hh).astype(_f32)


def _prea_body(xT_ref, z_ref, xgT_ref, zg_ref, lat_ref, wq_ref, wk_ref,
               idx_ref, den_ref, et_ref, eg_ref):
    xT = xT_ref[0]
    xgT = xgT_ref[0]
    z = z_ref[0]
    zg = zg_ref[0]
    lat = lat_ref[...]
    wq = wq_ref[...]
    wk = wk_ref[...]

    x0 = xT[0:1, :]
    x1 = xT[1:2, :]
    g0 = xgT[0:1, :]
    g1 = xgT[1:2, :]
    mn0 = jnp.min(g0, axis=1, keepdims=True)
    mx0 = jnp.max(g0, axis=1, keepdims=True)
    mn1 = jnp.min(g1, axis=1, keepdims=True)
    mx1 = jnp.max(g1, axis=1, keepdims=True)
    sp0 = (mx0 - mn0) / (G1 - 1.0)
    sp1 = (mx1 - mn1) / (G2 - 1.0)
    m0 = jnp.clip(jnp.floor((x0 - mn0 + 0.5 * sp0) / sp0), 0.0, G1 - 1.0)
    m1 = jnp.clip(jnp.floor((x1 - mn1 + 0.5 * sp1) / sp1), 0.0, G2 - 1.0)
    idx = (m0 * float(G2) + m1).astype(_i32)
    idx_ref[0] = idx

    q = jnp.dot(lat, wq, precision=_PREC)
    a_mat = jnp.dot(wk * q, _head_select_T(),
                    precision=_PREC) * (1.0 / math.sqrt(HDIM))

    _TT = (((0,), (1,)), ((), ()))
    lt = lax.dot_general(a_mat, z, _TT, precision=_PREC)
    lg = lax.dot_general(a_mat, zg, _TT, precision=_PREC)
    c = jnp.maximum(jnp.max(lt, axis=1, keepdims=True),
                    jnp.max(lg, axis=1, keepdims=True))
    et = jnp.exp(lt - c)
    eg = jnp.exp(lg - c)
    et_ref[0] = et
    eg_ref[0] = eg

    for bb in range(NGP // BINBLK):
        bins = lax.broadcasted_iota(_i32, (BINBLK, NTOK), 0) + bb * BINBLK
        maskf = (idx == bins).astype(_f32)
        den_blk = lax.dot_general(et, maskf, _NT, precision=_PREC)
        den_ref[0, :, pl.ds(bb * BINBLK, BINBLK)] = (
            den_blk + eg[:, bb * BINBLK:(bb + 1) * BINBLK])


def _full(shape):
    return pl.BlockSpec(shape, lambda m, _n=len(shape): (0,) * _n)


def _batch(shape):
    return pl.BlockSpec((1,) + shape, lambda m, _n=len(shape): (m,) + (0,) * _n)


def _prea_call(xT, z, xgT, zg, lat2, Wq, Wk):
    return pl.pallas_call(
        _prea_body,
        grid=(MB,),
        in_specs=[
            _batch((DX, NTOK)),
            _batch((NTOK, EMBED)),
            _batch((DX, NGP)),
            _batch((NGP, EMBED)),
            _full((1, EMBED)),
            _full((EMBED, EMBED)),
            _full((EMBED, EMBED)),
        ],
        out_specs=[
            _batch((1, NTOK)),
            _batch((HEADS, NGP)),
            _batch((HEADS, NTOK)),
            _batch((HEADS, NGP)),
        ],
        out_shape=[
            jax.ShapeDtypeStruct((MB, 1, NTOK), _i32),
            jax.ShapeDtypeStruct((MB, HEADS, NGP), _f32),
            jax.ShapeDtypeStruct((MB, HEADS, NTOK), _f32),
            jax.ShapeDtypeStruct((MB, HEADS, NGP), _f32),
        ],
    )(xT, z, xgT, zg, lat2, Wq, Wk)


def _preb_body(z_ref, e_ref, wv_ref, payload_ref):
    z = z_ref[0]
    e = e_ref[0]
    wv = wv_ref[...]
    vt = lax.dot_general(wv, z, (((0,), (1,)), ((), ())), precision=_PREC)
    ew = lax.dot_general(_head_select_T(), e, _NN, precision=_PREC)
    payload_ref[0] = (vt * ew).reshape(NCG, NLANE, NTOK)


def _preb_call(z, e, Wv):
    return pl.pallas_call(
        _preb_body,
        grid=(MB,),
        in_specs=[
            _batch((NTOK, EMBED)),
            _batch((HEADS, NTOK)),
            _full((EMBED, EMBED)),
        ],
        out_specs=_batch((NCG, NLANE, NTOK)),
        out_shape=jax.ShapeDtypeStruct((MB, NCG, NLANE, NTOK), _f32),
    )(z, e, Wv)


SCPAD = 1


def _sc_body(tokp_hbm, gridp_hbm, idx_hbm, out_hbm, acc_v, stage_v, irow_v):
    c = lax.axis_index("c")
    s = lax.axis_index("s")
    wid = c * NSUB + s
    for it in range(NITEM // NWORK):
        item = wid + it * NWORK
        b = item // NCG
        cg = item % NCG
        pltpu.sync_copy(gridp_hbm.at[b, cg], acc_v.at[:, pl.ds(0, NGP)])
        for t in range(NTOK // SCCHUNK):
            pltpu.sync_copy(tokp_hbm.at[b, cg, :, pl.ds(t * SCCHUNK, SCCHUNK)],
                            stage_v.at[:, pl.ds(0, SCCHUNK)])
            pltpu.sync_copy(idx_hbm.at[b, :, pl.ds(t * SCCHUNK, SCCHUNK)],
                            irow_v.at[:, pl.ds(0, SCCHUNK)])

            def step(i, carry):
                lane = lax.broadcasted_iota(_i32, (NLANE,), 0)
                fi = jnp.full((NLANE,), i, _i32)
                row = plsc.load_gather(irow_v, [lane, fi])
                val = plsc.load_gather(stage_v, [lane, fi])
                plsc.addupdate_scatter(acc_v, [lane, row], val)
                return carry

            lax.fori_loop(0, SCCHUNK, step, 0, unroll=8)
        pltpu.sync_copy(acc_v.at[:, pl.ds(0, NGP)], out_hbm.at[b, cg])


@functools.cache
def _sc_scatter():
    return pl.kernel(
        _sc_body,
        out_type=jax.ShapeDtypeStruct((MB, NCG, NLANE, NGP), _f32),
        compiler_params=pltpu.CompilerParams(needs_layout_passes=False),
        mesh=plsc.VectorSubcoreMesh(core_axis_name="c", subcore_axis_name="s"),
        scratch_types=[
            pltpu.VMEM((NLANE, NGP + SCPAD), _f32),
            pltpu.VMEM((NLANE, SCCHUNK + SCPAD), _f32),
            pltpu.VMEM((NLANE, SCCHUNK + SCPAD), _i32),
        ],
    )


def _post_body(acc_ref, den_ref, wo_ref, out_ref):
    acc = acc_ref[0].reshape(EMBED, NGP)
    den = den_ref[0]
    den_wide = lax.dot_general(_head_select_T(), den, _NN,
                               precision=_PREC)
    y = acc / den_wide
    out_ref[0] = lax.dot_general(y, wo_ref[...], _TN,
                                 precision=_PREC)


def _post_call(acc, den, Wo):
    return pl.pallas_call(
        _post_body,
        grid=(MB,),
        in_specs=[
            pl.BlockSpec((1, NCG, NLANE, NGP), lambda m: (m, 0, 0, 0)),
            pl.BlockSpec((1, HEADS, NGP), lambda m: (m, 0, 0)),
            pl.BlockSpec((EMBED, EMBED), lambda m: (0, 0)),
        ],
        out_specs=pl.BlockSpec((1, NGP, EMBED), lambda m: (m, 0, 0)),
        out_shape=jax.ShapeDtypeStruct((MB, NGP, EMBED), _f32),
    )(acc, den, Wo)


def kernel(x, z, x_grid, z_grid, latent, Wq, Wk, Wv, Wo):
    xT = x.transpose(0, 2, 1)
    xgT = x_grid.reshape(MB, NGP, DX).transpose(0, 2, 1)
    zg_flat = z_grid.reshape(MB, NGP, EMBED)
    lat2 = latent.reshape(1, EMBED)

    idx3, den, et, eg = _prea_call(xT, z, xgT, zg_flat, lat2, Wq, Wk)
    tokp = _preb_call(z, et, Wv)
    gridp = _preb_call(zg_flat, eg, Wv)
    idx_rep = jnp.broadcast_to(idx3, (MB, NLANE, NTOK))

    acc = _sc_scatter()(tokp, gridp, idx_rep)

    zout = _post_call(acc, den, Wo)
    return (x_grid, zout.reshape(MB, G1, G2, EMBED))


# speedup vs baseline: 10.3716x; 1.6640x over previous
import functools
import math

import jax
import jax.numpy as jnp
from jax import lax
from jax.experimental import pallas as pl
from jax.experimental.pallas import tpu as pltpu
from jax.experimental.pallas import tpu_sc as plsc

EMBED = 256
HEADS = 8
HDIM = 32
MB = 4
NTOK = 4096
G1 = 64
G2 = 64
NGP = G1 * G2
DX = 2
BINBLK = 256
NLANE = 16
NCG = EMBED // NLANE
NSUB = 16
NWORK = 32
NITEM = MB * NCG
SCCHUNK = 512

_PREC = lax.Precision.HIGHEST
_f32 = jnp.float32
_i32 = jnp.int32

_NN = (((1,), (0,)), ((), ()))
_NT = (((1,), (1,)), ((), ()))
_TN = (((0,), (0,)), ((), ()))


def _head_select_T():
    dd = lax.broadcasted_iota(_i32, (EMBED, HEADS), 0)
    hh = lax.broadcasted_iota(_i32, (EMBED, HEADS), 1)
    return (dd // HDIM == hh).astype(_f32)


def _prea_body(xT_ref, z_ref, xgT_ref, zg_ref, lat_ref, wq_ref, wk_ref,
               idx_ref, den_ref, et_ref, eg_ref):
    xT = xT_ref[0]
    xgT = xgT_ref[0]
    z = z_ref[0]
    zg = zg_ref[0]
    lat = lat_ref[...]
    wq = wq_ref[...]
    wk = wk_ref[...]

    x0 = xT[0:1, :]
    x1 = xT[1:2, :]
    g0 = xgT[0:1, :]
    g1 = xgT[1:2, :]
    mn0 = jnp.min(g0, axis=1, keepdims=True)
    mx0 = jnp.max(g0, axis=1, keepdims=True)
    mn1 = jnp.min(g1, axis=1, keepdims=True)
    mx1 = jnp.max(g1, axis=1, keepdims=True)
    sp0 = (mx0 - mn0) / (G1 - 1.0)
    sp1 = (mx1 - mn1) / (G2 - 1.0)
    m0 = jnp.clip(jnp.floor((x0 - mn0 + 0.5 * sp0) / sp0), 0.0, G1 - 1.0)
    m1 = jnp.clip(jnp.floor((x1 - mn1 + 0.5 * sp1) / sp1), 0.0, G2 - 1.0)
    idx = (m0 * float(G2) + m1).astype(_i32)
    idx_ref[0] = idx

    q = jnp.dot(lat, wq, precision=_PREC)
    a_mat = jnp.dot(wk * q, _head_select_T(),
                    precision=_PREC) * (1.0 / math.sqrt(HDIM))

    _TT = (((0,), (1,)), ((), ()))
    lt = lax.dot_general(a_mat, z, _TT, precision=_PREC)
    lg = lax.dot_general(a_mat, zg, _TT, precision=_PREC)
    c = jnp.maximum(jnp.max(lt, axis=1, keepdims=True),
                    jnp.max(lg, axis=1, keepdims=True))
    et = jnp.exp(lt - c)
    eg = jnp.exp(lg - c)
    et_ref[0] = et
    eg_ref[0] = eg

    et16 = et.astype(jnp.bfloat16)
    for bb in range(NGP // BINBLK):
        bins = lax.broadcasted_iota(_i32, (BINBLK, NTOK), 0) + bb * BINBLK
        mask16 = (idx == bins).astype(jnp.bfloat16)
        den_blk = lax.dot_general(et16, mask16, _NT,
                                  preferred_element_type=_f32)
        den_ref[0, :, pl.ds(bb * BINBLK, BINBLK)] = (
            den_blk + eg[:, bb * BINBLK:(bb + 1) * BINBLK])


def _full(shape):
    return pl.BlockSpec(shape, lambda m, _n=len(shape): (0,) * _n)


def _batch(shape):
    return pl.BlockSpec((1,) + shape, lambda m, _n=len(shape): (m,) + (0,) * _n)


def _prea_call(xT, z, xgT, zg, lat2, Wq, Wk):
    return pl.pallas_call(
        _prea_body,
        grid=(MB,),
        in_specs=[
            _batch((DX, NTOK)),
            _batch((NTOK, EMBED)),
            _batch((DX, NGP)),
            _batch((NGP, EMBED)),
            _full((1, EMBED)),
            _full((EMBED, EMBED)),
            _full((EMBED, EMBED)),
        ],
        out_specs=[
            _batch((1, NTOK)),
            _batch((HEADS, NGP)),
            _batch((HEADS, NTOK)),
            _batch((HEADS, NGP)),
        ],
        out_shape=[
            jax.ShapeDtypeStruct((MB, 1, NTOK), _i32),
            jax.ShapeDtypeStruct((MB, HEADS, NGP), _f32),
            jax.ShapeDtypeStruct((MB, HEADS, NTOK), _f32),
            jax.ShapeDtypeStruct((MB, HEADS, NGP), _f32),
        ],
    )(xT, z, xgT, zg, lat2, Wq, Wk)


def _preb_body(z_ref, e_ref, wv_ref, payload_ref):
    z = z_ref[0]
    e = e_ref[0]
    wv = wv_ref[...]
    vt = lax.dot_general(wv, z, (((0,), (1,)), ((), ())), precision=_PREC)
    ew = lax.dot_general(_head_select_T(), e, _NN, precision=_PREC)
    payload_ref[0] = (vt * ew).reshape(NCG, NLANE, NTOK)


def _preb_call(z, e, Wv):
    return pl.pallas_call(
        _preb_body,
        grid=(MB,),
        in_specs=[
            _batch((NTOK, EMBED)),
            _batch((HEADS, NTOK)),
            _full((EMBED, EMBED)),
        ],
        out_specs=_batch((NCG, NLANE, NTOK)),
        out_shape=jax.ShapeDtypeStruct((MB, NCG, NLANE, NTOK), _f32),
    )(z, e, Wv)


SCPAD = 1


def _sc_body(tokp_hbm, gridp_hbm, idx_hbm, out_hbm, acc_v, stage_v, irow_v):
    c = lax.axis_index("c")
    s = lax.axis_index("s")
    wid = c * NSUB + s
    for it in range(NITEM // NWORK):
        item = wid + it * NWORK
        b = item // NCG
        cg = item % NCG
        pltpu.sync_copy(gridp_hbm.at[b, cg], acc_v.at[:, pl.ds(0, NGP)])
        for t in range(NTOK // SCCHUNK):
            pltpu.sync_copy(tokp_hbm.at[b, cg, :, pl.ds(t * SCCHUNK, SCCHUNK)],
                            stage_v.at[:, pl.ds(0, SCCHUNK)])
            pltpu.sync_copy(
                idx_hbm.at[b, 0, pl.ds(t * SCCHUNK * NLANE, SCCHUNK * NLANE)],
                irow_v)

            def step(i, carry):
                lane = lax.broadcasted_iota(_i32, (NLANE,), 0)
                fi = jnp.full((NLANE,), i, _i32)
                row = irow_v[pl.ds(i * NLANE, NLANE)]
                val = plsc.load_gather(stage_v, [lane, fi])
                plsc.addupdate_scatter(acc_v, [lane, row], val)
                return carry

            lax.fori_loop(0, SCCHUNK, step, 0, unroll=8)
        pltpu.sync_copy(acc_v.at[:, pl.ds(0, NGP)], out_hbm.at[b, cg])


@functools.cache
def _sc_scatter():
    return pl.kernel(
        _sc_body,
        out_type=jax.ShapeDtypeStruct((MB, NCG, NLANE, NGP), _f32),
        compiler_params=pltpu.CompilerParams(needs_layout_passes=False),
        mesh=plsc.VectorSubcoreMesh(core_axis_name="c", subcore_axis_name="s"),
        scratch_types=[
            pltpu.VMEM((NLANE, NGP + SCPAD), _f32),
            pltpu.VMEM((NLANE, SCCHUNK + SCPAD), _f32),
            pltpu.VMEM((SCCHUNK * NLANE,), _i32),
        ],
    )


def _post_body(acc_ref, den_ref, wo_ref, out_ref):
    acc = acc_ref[0].reshape(EMBED, NGP)
    den = den_ref[0]
    den_wide = lax.dot_general(_head_select_T(), den, _NN,
                               precision=_PREC)
    y = acc / den_wide
    out_ref[0] = lax.dot_general(y, wo_ref[...], _TN,
                                 precision=_PREC)


def _post_call(acc, den, Wo):
    return pl.pallas_call(
        _post_body,
        grid=(MB,),
        in_specs=[
            pl.BlockSpec((1, NCG, NLANE, NGP), lambda m: (m, 0, 0, 0)),
            pl.BlockSpec((1, HEADS, NGP), lambda m: (m, 0, 0)),
            pl.BlockSpec((EMBED, EMBED), lambda m: (0, 0)),
        ],
        out_specs=pl.BlockSpec((1, NGP, EMBED), lambda m: (m, 0, 0)),
        out_shape=jax.ShapeDtypeStruct((MB, NGP, EMBED), _f32),
    )(acc, den, Wo)


def kernel(x, z, x_grid, z_grid, latent, Wq, Wk, Wv, Wo):
    xT = x.transpose(0, 2, 1)
    xgT = x_grid.reshape(MB, NGP, DX).transpose(0, 2, 1)
    zg_flat = z_grid.reshape(MB, NGP, EMBED)
    lat2 = latent.reshape(1, EMBED)

    idx3, den, et, eg = _prea_call(xT, z, xgT, zg_flat, lat2, Wq, Wk)
    tokp = _preb_call(z, et, Wv)
    gridp = _preb_call(zg_flat, eg, Wv)
    idx_il = jnp.broadcast_to(
        idx3.reshape(MB, NTOK, 1), (MB, NTOK, NLANE)
    ).reshape(MB, 1, NTOK * NLANE)

    acc = _sc_scatter()(tokp, gridp, idx_il)

    zout = _post_call(acc, den, Wo)
    return (x_grid, zout.reshape(MB, G1, G2, EMBED))


# speedup vs baseline: 19.4108x; 1.8715x over previous
import functools
import math

import jax
import jax.numpy as jnp
from jax import lax
from jax.experimental import pallas as pl
from jax.experimental.pallas import tpu as pltpu
from jax.experimental.pallas import tpu_sc as plsc

EMBED = 256
HEADS = 8
HDIM = 32
MB = 4
NTOK = 4096
G1 = 64
G2 = 64
NGP = G1 * G2
DX = 2
BINBLK = 256
NLANE = 16
NCG = EMBED // NLANE
NSUB = 16
NWORK = 32
NITEM = MB * NCG
SCCHUNK = 512

_PREC = lax.Precision.HIGHEST
_f32 = jnp.float32
_i32 = jnp.int32

_NN = (((1,), (0,)), ((), ()))
_NT = (((1,), (1,)), ((), ()))
_TN = (((0,), (0,)), ((), ()))


def _head_select_T():
    dd = lax.broadcasted_iota(_i32, (EMBED, HEADS), 0)
    hh = lax.broadcasted_iota(_i32, (EMBED, HEADS), 1)
    return (dd // HDIM == hh).astype(_f32)


def _prea_body(xT_ref, z_ref, xgT_ref, zg_ref, lat_ref, wq_ref, wk_ref,
               idx_ref, den_ref, et_ref, eg_ref):
    xT = xT_ref[0]
    xgT = xgT_ref[0]
    z = z_ref[0]
    zg = zg_ref[0]
    lat = lat_ref[...]
    wq = wq_ref[...]
    wk = wk_ref[...]

    x0 = xT[0:1, :]
    x1 = xT[1:2, :]
    g0 = xgT[0:1, :]
    g1 = xgT[1:2, :]
    mn0 = jnp.min(g0, axis=1, keepdims=True)
    mx0 = jnp.max(g0, axis=1, keepdims=True)
    mn1 = jnp.min(g1, axis=1, keepdims=True)
    mx1 = jnp.max(g1, axis=1, keepdims=True)
    sp0 = (mx0 - mn0) / (G1 - 1.0)
    sp1 = (mx1 - mn1) / (G2 - 1.0)
    m0 = jnp.clip(jnp.floor((x0 - mn0 + 0.5 * sp0) / sp0), 0.0, G1 - 1.0)
    m1 = jnp.clip(jnp.floor((x1 - mn1 + 0.5 * sp1) / sp1), 0.0, G2 - 1.0)
    idx = (m0 * float(G2) + m1).astype(_i32)
    idx_ref[0] = idx

    q = jnp.dot(lat, wq, precision=_PREC)
    a_mat = jnp.dot(wk * q, _head_select_T(),
                    precision=_PREC) * (1.0 / math.sqrt(HDIM))

    _TT = (((0,), (1,)), ((), ()))
    lt = lax.dot_general(a_mat, z, _TT, precision=_PREC)
    lg = lax.dot_general(a_mat, zg, _TT, precision=_PREC)
    c = jnp.maximum(jnp.max(lt, axis=1, keepdims=True),
                    jnp.max(lg, axis=1, keepdims=True))
    et = jnp.exp(lt - c)
    eg = jnp.exp(lg - c)
    et_ref[0] = et
    eg_ref[0] = eg

    et16 = et.astype(jnp.bfloat16)
    for bb in range(NGP // BINBLK):
        bins = lax.broadcasted_iota(_i32, (BINBLK, NTOK), 0) + bb * BINBLK
        mask16 = (idx == bins).astype(jnp.bfloat16)
        den_blk = lax.dot_general(et16, mask16, _NT,
                                  preferred_element_type=_f32)
        den_ref[0, :, pl.ds(bb * BINBLK, BINBLK)] = (
            den_blk + eg[:, bb * BINBLK:(bb + 1) * BINBLK])


def _full(shape):
    return pl.BlockSpec(shape, lambda m, _n=len(shape): (0,) * _n)


def _batch(shape):
    return pl.BlockSpec((1,) + shape, lambda m, _n=len(shape): (m,) + (0,) * _n)


def _prea_call(xT, z, xgT, zg, lat2, Wq, Wk):
    return pl.pallas_call(
        _prea_body,
        grid=(MB,),
        in_specs=[
            _batch((DX, NTOK)),
            _batch((NTOK, EMBED)),
            _batch((DX, NGP)),
            _batch((NGP, EMBED)),
            _full((1, EMBED)),
            _full((EMBED, EMBED)),
            _full((EMBED, EMBED)),
        ],
        out_specs=[
            _batch((1, NTOK)),
            _batch((HEADS, NGP)),
            _batch((HEADS, NTOK)),
            _batch((HEADS, NGP)),
        ],
        out_shape=[
            jax.ShapeDtypeStruct((MB, 1, NTOK), _i32),
            jax.ShapeDtypeStruct((MB, HEADS, NGP), _f32),
            jax.ShapeDtypeStruct((MB, HEADS, NTOK), _f32),
            jax.ShapeDtypeStruct((MB, HEADS, NGP), _f32),
        ],
    )(xT, z, xgT, zg, lat2, Wq, Wk)


def _preb_body(z_ref, e_ref, wv_ref, payload_ref):
    z = z_ref[0]
    e = e_ref[0]
    wv = wv_ref[...]
    vt = lax.dot_general(wv, z, (((0,), (1,)), ((), ())), precision=_PREC)
    ew = lax.dot_general(_head_select_T(), e, _NN, precision=_PREC)
    payload_ref[0] = (vt * ew).reshape(NCG, NLANE, NTOK)


def _preb_call(z, e, Wv):
    return pl.pallas_call(
        _preb_body,
        grid=(MB,),
        in_specs=[
            _batch((NTOK, EMBED)),
            _batch((HEADS, NTOK)),
            _full((EMBED, EMBED)),
        ],
        out_specs=_batch((NCG, NLANE, NTOK)),
        out_shape=jax.ShapeDtypeStruct((MB, NCG, NLANE, NTOK), _f32),
    )(z, e, Wv)


SCPAD = 1


def _sc_body(tokp_hbm, gridp_hbm, idx_hbm, out_hbm, acc_v,
             st0, st1, ir0, ir1, sem0, sem1):
    c = lax.axis_index("c")
    s = lax.axis_index("s")
    wid = c * NSUB + s
    stages = (st0, st1)
    irows = (ir0, ir1)
    sems = (sem0, sem1)
    nchk = NTOK // SCCHUNK
    for it in range(NITEM // NWORK):
        item = wid + it * NWORK
        b = item // NCG
        cg = item % NCG
        pltpu.sync_copy(gridp_hbm.at[b, cg], acc_v.at[:, pl.ds(0, NGP)])

        def start(t, buf):
            h1 = pltpu.async_copy(
                tokp_hbm.at[b, cg, :, pl.ds(t * SCCHUNK, SCCHUNK)],
                stages[buf].at[:, pl.ds(0, SCCHUNK)], sems[buf])
            h2 = pltpu.async_copy(
                idx_hbm.at[b, 0, pl.ds(t * SCCHUNK * NLANE, SCCHUNK * NLANE)],
                irows[buf], sems[buf])
            return h1, h2

        hs = start(0, 0)
        for t in range(nchk):
            buf = t % 2
            for h in hs:
                h.wait()
            if t + 1 < nchk:
                hs = start(t + 1, 1 - buf)
            stage_v = stages[buf]
            irow_v = irows[buf]

            @functools.partial(plsc.parallel_loop, 0, SCCHUNK, unroll=8)
            def body(i, _stage=stage_v, _irow=irow_v):
                lane = lax.broadcasted_iota(_i32, (NLANE,), 0)
                fi = jnp.full((NLANE,), i, _i32)
                row = _irow[pl.ds(i * NLANE, NLANE)]
                val = plsc.load_gather(_stage, [lane, fi])
                plsc.addupdate_scatter(acc_v, [lane, row], val)

        pltpu.sync_copy(acc_v.at[:, pl.ds(0, NGP)], out_hbm.at[b, cg])


@functools.cache
def _sc_scatter():
    return pl.kernel(
        _sc_body,
        out_type=jax.ShapeDtypeStruct((MB, NCG, NLANE, NGP), _f32),
        compiler_params=pltpu.CompilerParams(needs_layout_passes=False),
        mesh=plsc.VectorSubcoreMesh(core_axis_name="c", subcore_axis_name="s"),
        scratch_types=[
            pltpu.VMEM((NLANE, NGP + SCPAD), _f32),
            pltpu.VMEM((NLANE, SCCHUNK + SCPAD), _f32),
            pltpu.VMEM((NLANE, SCCHUNK + SCPAD), _f32),
            pltpu.VMEM((SCCHUNK * NLANE,), _i32),
            pltpu.VMEM((SCCHUNK * NLANE,), _i32),
            pltpu.SemaphoreType.DMA,
            pltpu.SemaphoreType.DMA,
        ],
    )


def _post_body(acc_ref, den_ref, wo_ref, out_ref):
    acc = acc_ref[0].reshape(EMBED, NGP)
    den = den_ref[0]
    den_wide = lax.dot_general(_head_select_T(), den, _NN,
                               precision=_PREC)
    y = acc / den_wide
    out_ref[0] = lax.dot_general(y, wo_ref[...], _TN,
                                 precision=_PREC)


def _post_call(acc, den, Wo):
    return pl.pallas_call(
        _post_body,
        grid=(MB,),
        in_specs=[
            pl.BlockSpec((1, NCG, NLANE, NGP), lambda m: (m, 0, 0, 0)),
            pl.BlockSpec((1, HEADS, NGP), lambda m: (m, 0, 0)),
            pl.BlockSpec((EMBED, EMBED), lambda m: (0, 0)),
        ],
        out_specs=pl.BlockSpec((1, NGP, EMBED), lambda m: (m, 0, 0)),
        out_shape=jax.ShapeDtypeStruct((MB, NGP, EMBED), _f32),
    )(acc, den, Wo)


def kernel(x, z, x_grid, z_grid, latent, Wq, Wk, Wv, Wo):
    xT = x.transpose(0, 2, 1)
    xgT = x_grid.reshape(MB, NGP, DX).transpose(0, 2, 1)
    zg_flat = z_grid.reshape(MB, NGP, EMBED)
    lat2 = latent.reshape(1, EMBED)

    idx3, den, et, eg = _prea_call(xT, z, xgT, zg_flat, lat2, Wq, Wk)
    tokp = _preb_call(z, et, Wv)
    gridp = _preb_call(zg_flat, eg, Wv)
    idx_il = jnp.broadcast_to(
        idx3.reshape(MB, NTOK, 1), (MB, NTOK, NLANE)
    ).reshape(MB, 1, NTOK * NLANE)

    acc = _sc_scatter()(tokp, gridp, idx_il)

    zout = _post_call(acc, den, Wo)
    return (x_grid, zout.reshape(MB, G1, G2, EMBED))
